# baseline (speedup 1.0000x reference)
import functools

import jax
import jax.numpy as jnp
from jax import lax
from jax.experimental import pallas as pl
from jax.experimental.pallas import tpu as pltpu
from jax.experimental.pallas import tpu_sc as plsc

L = 16
NC = 2
NS = 16
NW = NC * NS

IPG = 128


@functools.lru_cache(maxsize=None)
def _make_sc_lookup(B, S, V, D):
    assert D == NW
    n_streams = B // IPG
    mesh = plsc.VectorSubcoreMesh(core_axis_name="c", subcore_axis_name="s")

    @functools.partial(
        pl.kernel,
        mesh=mesh,
        out_type=(
            jax.ShapeDtypeStruct((S, D, B), jnp.float32),
            jax.ShapeDtypeStruct((S, B), jnp.float32),
        ),
        scratch_types=[
            pltpu.VMEM((B,), jnp.int32),
            pltpu.VMEM((B,), jnp.float32),
            pltpu.VMEM((B,), jnp.float32),
            pltpu.SemaphoreType.DMA,
        ],
        compiler_params=pltpu.CompilerParams(use_tc_tiling_on_sc=False),
    )
    def k(tabT_hbm, idxT_hbm, resT_hbm, maskT_hbm, idx_v, col_v, msk_v, sem):
        w = lax.axis_index("s") * NC + lax.axis_index("c")
        tcol = tabT_hbm.at[w]

        def stripe_body(s, carry):
            pltpu.sync_copy(idxT_hbm.at[s], idx_v)
            copies = []
            for j in range(n_streams):
                src = tcol.at[idx_v.at[pl.ds(j * IPG, IPG)]]
                dst = col_v.at[pl.ds(j * IPG, IPG)]
                copies.append(pltpu.async_copy(src, dst, sem))
            for c in copies:
                c.wait()
            pltpu.sync_copy(col_v, resT_hbm.at[s, w])
            return carry

        lax.fori_loop(0, S, stripe_body, 0)

        def mask_body(s):
            pltpu.sync_copy(idxT_hbm.at[s], idx_v)

            def cmp(i, c2):
                v = idx_v[pl.ds(i * L, L)]
                msk_v[pl.ds(i * L, L)] = jnp.where(
                    v == 0, jnp.float32(1.0), jnp.float32(0.0))
                return c2

            lax.fori_loop(0, B // L, cmp, 0)
            pltpu.sync_copy(msk_v, maskT_hbm.at[s])

        n_full = S // NW
        n_rem = S - n_full * NW

        def mask_outer(t, carry):
            mask_body(t * NW + w)
            return carry

        lax.fori_loop(0, n_full, mask_outer, 0)

        @pl.when(w < n_rem)
        def _():
            mask_body(n_full * NW + w)

    return k


def kernel(input, table):
    B, S = input.shape
    V, D = table.shape
    idxT = jnp.transpose(input)
    tabT = jnp.transpose(table)
    resT, maskT = _make_sc_lookup(B, S, V, D)(tabT, idxT)
    res = jnp.transpose(resT, (2, 0, 1))
    mask = jnp.transpose(maskT)
    return res, mask


# speedup vs baseline: 2.4175x; 2.4175x over previous
import functools

import jax
import jax.numpy as jnp
from jax import lax
from jax.experimental import pallas as pl
from jax.experimental.pallas import tpu as pltpu
from jax.experimental.pallas import tpu_sc as plsc

L = 16
NC = 2
NS = 16
NW = NC * NS

TW = 128


def _wid():
    return lax.axis_index("s") * NC + lax.axis_index("c")


@functools.lru_cache(maxsize=None)
def _make_relayout(V, D):
    n_full = V // TW
    rem = V - n_full * TW
    n_iter = n_full // NW
    n_extra = n_full - n_iter * NW
    mesh = plsc.VectorSubcoreMesh(core_axis_name="c", subcore_axis_name="s")

    @functools.partial(
        pl.kernel,
        mesh=mesh,
        out_type=jax.ShapeDtypeStruct((V * D,), jnp.float32),
        scratch_types=[
            pltpu.VMEM((D, TW), jnp.float32),
            pltpu.VMEM((TW * D,), jnp.float32),
        ],
        compiler_params=pltpu.CompilerParams(use_tc_tiling_on_sc=True, needs_layout_passes=False),
    )
    def k(tabT_hbm, tailT_hbm, lin_hbm, tile_v, out_v):
        w = _wid()
        lane = lax.broadcasted_iota(jnp.int32, (L,), 0)

        def transpose_block(width):
            def b_body(b, carry):
                col = jnp.full((L,), 0, jnp.int32) + b
                for h in range(D // L):
                    v = plsc.load_gather(tile_v, [lane + (h * L), col])
                    out_v[pl.ds(b * D + h * L, L)] = v
                return carry

            lax.fori_loop(0, width, b_body, 0)

        def do_block(j):
            pltpu.sync_copy(tabT_hbm.at[:, pl.ds(j * TW, TW)], tile_v)
            transpose_block(TW)
            pltpu.sync_copy(out_v, lin_hbm.at[pl.ds(j * (TW * D), TW * D)])

        def blk(t, carry):
            do_block(t * NW + w)
            return carry

        lax.fori_loop(0, n_iter, blk, 0)

        @pl.when(w < n_extra)
        def _():
            do_block(n_iter * NW + w)

        if rem:
            @pl.when(w == NW - 1)
            def _():
                pltpu.sync_copy(tailT_hbm, tile_v)
                transpose_block(TW)
                pltpu.sync_copy(out_v,
                                lin_hbm.at[pl.ds((V - TW) * D, TW * D)])

    return k


@functools.lru_cache(maxsize=None)
def _make_gather(B, S, V, D):
    nj = B // TW
    assert nj == NW and S % 2 == 0
    mesh = plsc.VectorSubcoreMesh(core_axis_name="c", subcore_axis_name="s")

    @functools.partial(
        pl.kernel,
        mesh=mesh,
        out_type=(
            jax.ShapeDtypeStruct((S, D // 8, nj, 8, TW), jnp.float32),
            jax.ShapeDtypeStruct((S, B), jnp.float32),
        ),
        scratch_types=[
            pltpu.VMEM((S, TW), jnp.int32),
            pltpu.VMEM((2, TW, D), jnp.float32),
            pltpu.VMEM((D // 8, 8, TW), jnp.float32),
            pltpu.VMEM((B,), jnp.int32),
            pltpu.VMEM((B,), jnp.float32),
            pltpu.SemaphoreType.DMA,
            pltpu.SemaphoreType.DMA,
            pltpu.SemaphoreType.DMA,
        ],
        compiler_params=pltpu.CompilerParams(
            use_tc_tiling_on_sc=False, needs_layout_passes=False),
    )
    def k(tab_hbm, idxT_hbm, res5_hbm, maskT_hbm,
          idx_v, rows_v, out_v, midx_v, msk_v, sem0, sem1, osem):
        w = _wid()
        lane = lax.broadcasted_iota(jnp.int32, (L,), 0)
        sems = (sem0, sem1)

        pltpu.sync_copy(idxT_hbm.at[:, pl.ds(w * TW, TW)], idx_v)

        def fire(s, buf):
            return pltpu.async_copy(
                tab_hbm.at[idx_v.at[s]], rows_v.at[buf], sems[buf])

        def transpose_and_write(s, buf):
            def c_loop(ci, carry):
                for i in range(D // 8):
                    col = jnp.full((L,), 0, jnp.int32) + (i * 8 + ci)
                    for h in range(TW // L):
                        v = plsc.load_gather(
                            rows_v.at[buf], [lane + (h * L), col])
                        out_v[i, ci, pl.ds(h * L, L)] = v
                return carry

            lax.fori_loop(0, 8, c_loop, 0)
            copies = [pltpu.async_copy(out_v.at[i], res5_hbm.at[s, i, w], osem)
                      for i in range(D // 8)]
            for cp in copies:
                cp.wait()

        fire(0, 0)

        def pair_body(t, carry):
            sA = t * 2
            sB = sA + 1
            cB = fire(sB, 1)
            pltpu.make_async_copy(
                tab_hbm.at[idx_v.at[sA]], rows_v.at[0], sems[0]).wait()
            transpose_and_write(sA, 0)

            @pl.when(sB + 1 < S)
            def _():
                fire(sB + 1, 0)

            cB.wait()
            transpose_and_write(sB, 1)
            return carry

        lax.fori_loop(0, S // 2, pair_body, 0)

        def mask_body(s):
            pltpu.sync_copy(idxT_hbm.at[s], midx_v)

            def cmp(i, c2):
                v = midx_v[pl.ds(i * L, L)]
                msk_v[pl.ds(i * L, L)] = jnp.where(
                    v == 0, jnp.float32(1.0), jnp.float32(0.0))
                return c2

            lax.fori_loop(0, B // L, cmp, 0)
            pltpu.sync_copy(msk_v, maskT_hbm.at[s])

        m_full = S // NW
        m_rem = S - m_full * NW

        def mask_outer(t, carry):
            mask_body(t * NW + w)
            return carry

        lax.fori_loop(0, m_full, mask_outer, 0)

        @pl.when(w < m_rem)
        def _():
            mask_body(m_full * NW + w)

    return k


def kernel(input, table):
    B, S = input.shape
    V, D = table.shape
    idxT = jnp.transpose(input)
    tabT = jnp.transpose(table)
    tailT = jnp.transpose(table[V - TW:])
    tab_lin = _make_relayout(V, D)(tabT, tailT).reshape(V, D)
    res5, maskT = _make_gather(B, S, V, D)(tab_lin, idxT)
    res = jnp.transpose(res5, (2, 4, 0, 1, 3)).reshape(B, S, D)
    mask = jnp.transpose(maskT)
    return res, mask


# speedup vs baseline: 2.9671x; 1.2274x over previous
import functools

import jax
import jax.numpy as jnp
from jax import lax
from jax.experimental import pallas as pl
from jax.experimental.pallas import tpu as pltpu
from jax.experimental.pallas import tpu_sc as plsc

L = 16
NC = 2
NS = 16
NW = NC * NS

TW = 128


def _wid():
    return lax.axis_index("s") * NC + lax.axis_index("c")


@functools.lru_cache(maxsize=None)
def _make_relayout(V, D):
    n_full = V // TW
    rem = V - n_full * TW
    n_iter = n_full // NW
    n_extra = n_full - n_iter * NW
    mesh = plsc.VectorSubcoreMesh(core_axis_name="c", subcore_axis_name="s")

    @functools.partial(
        pl.kernel,
        mesh=mesh,
        out_type=jax.ShapeDtypeStruct((V * D,), jnp.float32),
        scratch_types=[
            pltpu.VMEM((2, D, TW), jnp.float32),
            pltpu.VMEM((2, TW * D), jnp.float32),
            pltpu.SemaphoreType.DMA,
            pltpu.SemaphoreType.DMA,
            pltpu.SemaphoreType.DMA,
            pltpu.SemaphoreType.DMA,
        ],
        compiler_params=pltpu.CompilerParams(use_tc_tiling_on_sc=True, needs_layout_passes=False),
    )
    def k(tabT_hbm, tailT_hbm, lin_hbm, tile_v, out_v, is0, is1, os0, os1):
        w = _wid()
        lane = lax.broadcasted_iota(jnp.int32, (L,), 0)
        isems = (is0, is1)
        osems = (os0, os1)
        assert n_iter % 2 == 0

        def fire_in(j, p):
            pltpu.async_copy(tabT_hbm.at[:, pl.ds(j * TW, TW)],
                             tile_v.at[p], isems[p])

        def wait_in(p):
            pltpu.make_async_copy(tabT_hbm.at[:, pl.ds(0, TW)],
                                  tile_v.at[p], isems[p]).wait()

        def fire_out(j, p):
            pltpu.async_copy(out_v.at[p],
                             lin_hbm.at[pl.ds(j * (TW * D), TW * D)], osems[p])

        def wait_out(p):
            pltpu.make_async_copy(out_v.at[p],
                                  lin_hbm.at[pl.ds(0, TW * D)], osems[p]).wait()

        def transpose_block(p):
            def b_body(b, carry):
                col = jnp.full((L,), 0, jnp.int32) + b
                for h in range(D // L):
                    v = plsc.load_gather(tile_v.at[p], [lane + (h * L), col])
                    out_v[p, pl.ds(b * D + h * L, L)] = v
                return carry

            lax.fori_loop(0, TW, b_body, 0)

        def stage(t, jA, p):
            wait_in(p)

            @pl.when(t > 0)
            def _():
                wait_out(p)

            transpose_block(p)
            fire_out(jA, p)

        fire_in(w, 0)

        def pair(t, carry):
            jA = (2 * t) * NW + w
            jB = jA + NW
            fire_in(jB, 1)
            stage(t, jA, 0)

            @pl.when(2 * t + 2 < n_iter)
            def _():
                fire_in(jA + 2 * NW, 0)

            stage(t, jB, 1)
            return carry

        lax.fori_loop(0, n_iter // 2, pair, 0)
        wait_out(0)
        wait_out(1)

        @pl.when(w < n_extra)
        def _():
            j = n_iter * NW + w
            pltpu.sync_copy(tabT_hbm.at[:, pl.ds(j * TW, TW)], tile_v.at[0])
            transpose_block(0)
            pltpu.sync_copy(out_v.at[0],
                            lin_hbm.at[pl.ds(j * (TW * D), TW * D)])

        if rem:
            @pl.when(w == NW - 1)
            def _():
                pltpu.sync_copy(tailT_hbm, tile_v.at[1])
                transpose_block(1)
                pltpu.sync_copy(out_v.at[1],
                                lin_hbm.at[pl.ds((V - TW) * D, TW * D)])

    return k


@functools.lru_cache(maxsize=None)
def _make_gather(B, S, V, D):
    nj = B // TW
    assert nj == NW and S % 2 == 0
    mesh = plsc.VectorSubcoreMesh(core_axis_name="c", subcore_axis_name="s")

    @functools.partial(
        pl.kernel,
        mesh=mesh,
        out_type=(
            jax.ShapeDtypeStruct((S, D // 8, nj, 8, TW), jnp.float32),
            jax.ShapeDtypeStruct((S, B), jnp.float32),
        ),
        scratch_types=[
            pltpu.VMEM((S, TW), jnp.int32),
            pltpu.VMEM((2, TW, D), jnp.float32),
            pltpu.VMEM((2, D // 8, 8, TW), jnp.float32),
            pltpu.VMEM((B,), jnp.int32),
            pltpu.VMEM((B,), jnp.float32),
            pltpu.SemaphoreType.DMA,
            pltpu.SemaphoreType.DMA,
            pltpu.SemaphoreType.DMA,
            pltpu.SemaphoreType.DMA,
        ],
        compiler_params=pltpu.CompilerParams(
            use_tc_tiling_on_sc=False, needs_layout_passes=False),
    )
    def k(tab_hbm, idxT_hbm, res5_hbm, maskT_hbm,
          idx_v, rows_v, out_v, midx_v, msk_v, sem0, sem1, os0, os1):
        w = _wid()
        lane = lax.broadcasted_iota(jnp.int32, (L,), 0)
        sems = (sem0, sem1)
        osems = (os0, os1)

        pltpu.sync_copy(idxT_hbm.at[:, pl.ds(w * TW, TW)], idx_v)

        def fire(s, buf):
            return pltpu.async_copy(
                tab_hbm.at[idx_v.at[s]], rows_v.at[buf], sems[buf])

        def wait_outs(p):
            for i in range(D // 8):
                pltpu.make_async_copy(
                    out_v.at[p, i], res5_hbm.at[0, i, w], osems[p]).wait()

        def stage(t, s, p):
            pltpu.make_async_copy(
                tab_hbm.at[idx_v.at[s]], rows_v.at[p], sems[p]).wait()

            @pl.when(t > 0)
            def _():
                wait_outs(p)

            def c_loop(ci, carry):
                for i in range(D // 8):
                    col = jnp.full((L,), 0, jnp.int32) + (i * 8 + ci)
                    for h in range(TW // L):
                        v = plsc.load_gather(
                            rows_v.at[p], [lane + (h * L), col])
                        out_v[p, i, ci, pl.ds(h * L, L)] = v
                return carry

            lax.fori_loop(0, 8, c_loop, 0)
            for i in range(D // 8):
                pltpu.async_copy(out_v.at[p, i], res5_hbm.at[s, i, w],
                                 osems[p])

        fire(0, 0)

        def pair_body(t, carry):
            sA = t * 2
            sB = sA + 1
            fire(sB, 1)
            stage(t, sA, 0)

            @pl.when(sB + 1 < S)
            def _():
                fire(sB + 1, 0)

            stage(t, sB, 1)
            return carry

        lax.fori_loop(0, S // 2, pair_body, 0)
        wait_outs(0)
        wait_outs(1)

        def mask_body(s):
            pltpu.sync_copy(idxT_hbm.at[s], midx_v)

            def cmp(i, c2):
                v = midx_v[pl.ds(i * L, L)]
                msk_v[pl.ds(i * L, L)] = jnp.where(
                    v == 0, jnp.float32(1.0), jnp.float32(0.0))
                return c2

            lax.fori_loop(0, B // L, cmp, 0)
            pltpu.sync_copy(msk_v, maskT_hbm.at[s])

        m_full = S // NW
        m_rem = S - m_full * NW

        def mask_outer(t, carry):
            mask_body(t * NW + w)
            return carry

        lax.fori_loop(0, m_full, mask_outer, 0)

        @pl.when(w < m_rem)
        def _():
            mask_body(m_full * NW + w)

    return k


def kernel(input, table):
    B, S = input.shape
    V, D = table.shape
    idxT = jnp.transpose(input)
    tabT = jnp.transpose(table)
    tailT = jnp.transpose(table[V - TW:])
    tab_lin = _make_relayout(V, D)(tabT, tailT).reshape(V, D)
    res5, maskT = _make_gather(B, S, V, D)(tab_lin, idxT)
    res = jnp.transpose(res5, (2, 4, 0, 1, 3)).reshape(B, S, D)
    mask = jnp.transpose(maskT)
    return res, mask
